# baseline (speedup 1.0000x reference)
import functools

import jax
import jax.numpy as jnp
from jax import lax
from jax.experimental import pallas as pl
from jax.experimental.pallas import tpu as pltpu
from jax.experimental.pallas import tpu_sc as plsc

C = 80
GY, GX = 128, 128
NUM_POINTS = 249216
NUM_GRIDS = GY * GX

NC, NS, L = 2, 16, 16
NW = NC * NS
UNROLL = 4
CHUNK = 7808
NPAD = NW * CHUNK
NVREG = CHUNK // L
PRIV = 16512
VPT = NUM_GRIDS // NW

_mesh = plsc.VectorSubcoreMesh(
    core_axis_name="c", subcore_axis_name="s", num_cores=NC, num_subcores=NS
)


def _wid():
    return lax.axis_index("s") * NC + lax.axis_index("c")


@functools.partial(
    pl.kernel,
    out_type=jax.ShapeDtypeStruct((NW * NUM_GRIDS,), jnp.int32),
    mesh=_mesh,
    scratch_types=[
        pltpu.VMEM((CHUNK,), jnp.int32),
        pltpu.VMEM((PRIV,), jnp.int32),
        pltpu.VMEM((UNROLL * L,), jnp.int32),
    ],
    compiler_params=pltpu.CompilerParams(needs_layout_passes=False),
)
def _k1_scatter_max(coor_hbm, win_hbm, cvm, priv, sh):
    wid = _wid()
    base = wid * CHUNK
    pltpu.sync_copy(coor_hbm.at[pl.ds(base, CHUNK)], cvm)

    iota = lax.iota(jnp.int32, L)
    neg1 = jnp.full((L,), -1, jnp.int32)

    def init_body(i, carry):
        for u in range(8):
            priv[pl.ds((i * 8 + u) * L, L)] = neg1
        return carry

    lax.fori_loop(0, PRIV // (8 * L), init_body, 0)

    shm1 = jnp.maximum(iota - 1, 0)
    mge1 = iota >= 1
    is_last_lane = iota == L - 1

    def body(k, carry):
        for u in range(UNROLL):
            kk = k * UNROLL + u
            c16 = cvm[pl.ds(kk * L, L)]
            gidx = base + kk * L + iota
            key = c16 * L + iota
            ks, vs = plsc.sort_key_val(key, gidx)
            cs = lax.shift_right_logical(ks, 4)
            plsc.store_scatter(sh, [shm1 + u * L], cs, mask=mge1)
            nxt = sh[pl.ds(u * L, L)]
            last = jnp.logical_or(cs != nxt, is_last_lane)
            plsc.store_scatter(priv, [cs], vs, mask=last)
        return carry

    lax.fori_loop(0, NVREG // UNROLL, body, 0)
    pltpu.sync_copy(priv.at[pl.ds(0, NUM_GRIDS)],
                    win_hbm.at[pl.ds(wid * NUM_GRIDS, NUM_GRIDS)])


@functools.partial(
    pl.kernel,
    out_type=jax.ShapeDtypeStruct((NUM_GRIDS,), jnp.int32),
    mesh=_mesh,
    scratch_types=[
        pltpu.VMEM((NW * VPT,), jnp.int32),
        pltpu.VMEM((VPT,), jnp.int32),
        pltpu.SemaphoreType.DMA,
    ],
    compiler_params=pltpu.CompilerParams(needs_layout_passes=False),
)
def _k2a_merge(win_hbm, winner_hbm, slab, wvm, sem):
    wid = _wid()
    vbase = wid * VPT

    for t in range(NW):
        pltpu.async_copy(win_hbm.at[pl.ds(t * NUM_GRIDS + vbase, VPT)],
                         slab.at[pl.ds(t * VPT, VPT)], sem)
    pltpu.make_async_copy(win_hbm.at[pl.ds(0, NW * VPT)], slab, sem).wait()

    def merge_body(j, carry):
        m = slab[pl.ds(j * L, L)]
        for t in range(1, NW):
            m = jnp.maximum(m, slab[pl.ds(t * VPT + j * L, L)])
        wvm[pl.ds(j * L, L)] = m
        return carry

    lax.fori_loop(0, VPT // L, merge_body, 0)
    pltpu.sync_copy(wvm, winner_hbm.at[pl.ds(vbase, VPT)])


NWIN = CHUNK // 128


@functools.partial(
    pl.kernel,
    out_type=jax.ShapeDtypeStruct((NUM_GRIDS, C), jnp.float32),
    mesh=_mesh,
    scratch_types=[
        pltpu.VMEM((NUM_GRIDS,), jnp.int32),
        pltpu.VMEM((CHUNK,), jnp.int32),
        pltpu.VMEM((C, 128), jnp.float32),
        pltpu.VMEM((C, 128), jnp.float32),
        pltpu.VMEM((128, C), jnp.float32),
        pltpu.SemaphoreType.DMA,
        pltpu.SemaphoreType.DMA,
        pltpu.SemaphoreType.DMA,
    ],
    compiler_params=pltpu.CompilerParams(needs_layout_passes=False),
)
def _k2b_stream_select(winner_hbm, featT_hbm, g_hbm, wfull, dvm, bufa,
                       bufb, slots, sema, semb, gsem):
    wid = _wid()
    base = wid * CHUNK
    iota = lax.iota(jnp.int32, L)
    neg1 = jnp.full((L,), -1, jnp.int32)

    pltpu.sync_copy(winner_hbm, wfull)

    def dinit(i, carry):
        for u in range(8):
            dvm[pl.ds((i * 8 + u) * L, L)] = neg1
        return carry

    lax.fori_loop(0, NVREG // 8, dinit, 0)

    def dscan(j, carry):
        w16 = wfull[pl.ds(j * L, L)]
        rel = w16 - base
        m = jnp.logical_and(rel >= 0, rel < CHUNK)
        idx = jnp.where(m, rel, 0)
        plsc.store_scatter(dvm, [idx], j * L + iota, mask=m)
        return carry

    lax.fori_loop(0, NUM_GRIDS // L, dscan, 0)

    rows16 = [iota + 16 * b for b in range(C // L)]

    def fire(w, buf, sem):
        col = base + w * 128

        @pl.when(col < NUM_POINTS)
        def _():
            for g in range(C // 8):
                pltpu.async_copy(
                    featT_hbm.at[pl.ds(8 * g, 8), pl.ds(col, 128)],
                    buf.at[pl.ds(8 * g, 8)], sem)

    def drain(w, buf, sem):
        @pl.when(base + w * 128 < NUM_POINTS)
        def _():
            pltpu.make_async_copy(featT_hbm.at[pl.ds(0, C), pl.ds(0, 128)],
                                  buf, sem).wait()

    def process(w, buf):
        def jv_body(jv, wcnt):
            dvec = dvm[pl.ds(w * 128 + jv * L, L)]
            mask = dvec >= 0

            def cond(state):
                m, _ = state
                return jnp.any(m)

            def body(state):
                m, cnt = state
                lsp = plsc.all_reduce_ffs(m)
                pidx = w * 128 + jv * L + lsp
                v = plsc.load_gather(dvm, [pidx])[0]
                pcol = jv * L + lsp
                for b in range(C // L):
                    vals = plsc.load_gather(buf, [rows16[b], pcol])
                    plsc.store_scatter(
                        slots, [cnt + iota * 0, 16 * b + iota], vals)
                pltpu.async_copy(slots.at[pl.ds(cnt, 1)],
                                 g_hbm.at[pl.ds(v, 1)], gsem)
                return jnp.logical_and(m, iota != lsp), cnt + 1

            m2, wcnt = lax.while_loop(cond, body, (mask, wcnt))
            return wcnt

        wcnt = lax.fori_loop(0, 128 // L, jv_body, 0)

        def gdrain(i, carry):
            pltpu.make_async_copy(
                winner_hbm.at[pl.ds(0, C)], wfull.at[pl.ds(0, C)],
                gsem).wait()
            return carry

        lax.fori_loop(0, wcnt, gdrain, 0)

    fire(0, bufa, sema)

    def wloop(j, carry):
        fire(2 * j + 1, bufb, semb)
        drain(2 * j, bufa, sema)
        process(2 * j, bufa)
        fire(2 * j + 2, bufa, sema)
        drain(2 * j + 1, bufb, semb)
        process(2 * j + 1, bufb)
        return carry

    lax.fori_loop(0, (NWIN - 1) // 2, wloop, 0)
    drain(NWIN - 1, bufa, sema)
    process(NWIN - 1, bufa)


def _k3_body(g_ref, w_ref, o_ref):
    g = g_ref[...]
    w = w_ref[0]
    gt = jnp.transpose(g, (1, 0))
    o_ref[...] = jnp.where(w >= 0, gt, jnp.float32(0.0))


_NBLK = 16
_BV = NUM_GRIDS // _NBLK

_k3_transpose = pl.pallas_call(
    _k3_body,
    grid=(_NBLK,),
    in_specs=[
        pl.BlockSpec((_BV, C), lambda i: (i, 0)),
        pl.BlockSpec((1, 1, _BV), lambda i: (i, 0, 0)),
    ],
    out_specs=pl.BlockSpec((C, _BV), lambda i: (0, i)),
    out_shape=jax.ShapeDtypeStruct((C, NUM_GRIDS), jnp.float32),
)


@jax.jit
def kernel(feat, bev_feat, lidar_coor_1d):
    del bev_feat
    coor = lidar_coor_1d.astype(jnp.int32)
    pad = jnp.full((NPAD - NUM_POINTS,), NUM_GRIDS, jnp.int32)
    coor = jnp.concatenate([coor, pad])
    winners = _k1_scatter_max(coor)
    winner = _k2a_merge(winners)
    g = _k2b_stream_select(winner, jnp.transpose(feat))
    out = _k3_transpose(g, winner.reshape(_NBLK, 1, _BV))
    return out.reshape(1, C, GY, GX)


# speedup vs baseline: 3.2692x; 3.2692x over previous
import functools

import jax
import jax.numpy as jnp
from jax import lax
from jax.experimental import pallas as pl
from jax.experimental.pallas import tpu as pltpu
from jax.experimental.pallas import tpu_sc as plsc

C = 80
GY, GX = 128, 128
NUM_POINTS = 249216
NUM_GRIDS = GY * GX

NC, NS, L = 2, 16, 16
NW = NC * NS
UNROLL = 4
CHUNK = 7808
NPAD = NW * CHUNK
NVREG = CHUNK // L
PRIV = 16512
VPT = NUM_GRIDS // NW

_mesh = plsc.VectorSubcoreMesh(
    core_axis_name="c", subcore_axis_name="s", num_cores=NC, num_subcores=NS
)


def _wid():
    return lax.axis_index("s") * NC + lax.axis_index("c")


@functools.partial(
    pl.kernel,
    out_type=jax.ShapeDtypeStruct((NW * NUM_GRIDS,), jnp.int32),
    mesh=_mesh,
    scratch_types=[
        pltpu.VMEM((CHUNK,), jnp.int32),
        pltpu.VMEM((PRIV,), jnp.int32),
        pltpu.VMEM((UNROLL * L,), jnp.int32),
    ],
    compiler_params=pltpu.CompilerParams(needs_layout_passes=False),
)
def _k1_scatter_max(coor_hbm, win_hbm, cvm, priv, sh):
    wid = _wid()
    base = wid * CHUNK
    pltpu.sync_copy(coor_hbm.at[pl.ds(base, CHUNK)], cvm)

    iota = lax.iota(jnp.int32, L)
    neg1 = jnp.full((L,), -1, jnp.int32)

    def init_body(i, carry):
        for u in range(8):
            priv[pl.ds((i * 8 + u) * L, L)] = neg1
        return carry

    lax.fori_loop(0, PRIV // (8 * L), init_body, 0)

    shm1 = jnp.maximum(iota - 1, 0)
    mge1 = iota >= 1
    is_last_lane = iota == L - 1

    def body(k, carry):
        for u in range(UNROLL):
            kk = k * UNROLL + u
            c16 = cvm[pl.ds(kk * L, L)]
            gidx = base + kk * L + iota
            key = c16 * L + iota
            ks, vs = plsc.sort_key_val(key, gidx)
            cs = lax.shift_right_logical(ks, 4)
            plsc.store_scatter(sh, [shm1 + u * L], cs, mask=mge1)
            nxt = sh[pl.ds(u * L, L)]
            last = jnp.logical_or(cs != nxt, is_last_lane)
            plsc.store_scatter(priv, [cs], vs, mask=last)
        return carry

    lax.fori_loop(0, NVREG // UNROLL, body, 0)
    pltpu.sync_copy(priv.at[pl.ds(0, NUM_GRIDS)],
                    win_hbm.at[pl.ds(wid * NUM_GRIDS, NUM_GRIDS)])


@functools.partial(
    pl.kernel,
    out_type=jax.ShapeDtypeStruct((NUM_GRIDS,), jnp.int32),
    mesh=_mesh,
    scratch_types=[
        pltpu.VMEM((NW * VPT,), jnp.int32),
        pltpu.VMEM((VPT,), jnp.int32),
        pltpu.SemaphoreType.DMA,
    ],
    compiler_params=pltpu.CompilerParams(needs_layout_passes=False),
)
def _k2a_merge(win_hbm, winner_hbm, slab, wvm, sem):
    wid = _wid()
    vbase = wid * VPT

    for t in range(NW):
        pltpu.async_copy(win_hbm.at[pl.ds(t * NUM_GRIDS + vbase, VPT)],
                         slab.at[pl.ds(t * VPT, VPT)], sem)
    pltpu.make_async_copy(win_hbm.at[pl.ds(0, NW * VPT)], slab, sem).wait()

    def merge_body(j, carry):
        m = slab[pl.ds(j * L, L)]
        for t in range(1, NW):
            m = jnp.maximum(m, slab[pl.ds(t * VPT + j * L, L)])
        wvm[pl.ds(j * L, L)] = m
        return carry

    lax.fori_loop(0, VPT // L, merge_body, 0)
    pltpu.sync_copy(wvm, winner_hbm.at[pl.ds(vbase, VPT)])


NWIN = CHUNK // 128


@functools.partial(
    pl.kernel,
    out_type=jax.ShapeDtypeStruct((NUM_GRIDS, C), jnp.float32),
    mesh=_mesh,
    scratch_types=[
        pltpu.VMEM((NUM_GRIDS,), jnp.int32),
        pltpu.VMEM((CHUNK,), jnp.int32),
        pltpu.VMEM((C, 128), jnp.float32),
        pltpu.VMEM((C, 128), jnp.float32),
        pltpu.VMEM((128, C), jnp.float32),
        pltpu.SemaphoreType.DMA,
        pltpu.SemaphoreType.DMA,
        pltpu.SemaphoreType.DMA,
    ],
    compiler_params=pltpu.CompilerParams(needs_layout_passes=False),
)
def _k2b_stream_select(winner_hbm, featT_hbm, g_hbm, wfull, dvm, bufa,
                       bufb, slots, sema, semb, gsem):
    wid = _wid()
    iota = lax.iota(jnp.int32, L)
    neg1 = jnp.full((L,), -1, jnp.int32)

    pltpu.sync_copy(winner_hbm, wfull)

    def dinit(i, carry):
        for u in range(8):
            dvm[pl.ds((i * 8 + u) * L, L)] = neg1
        return carry

    lax.fori_loop(0, NVREG // 8, dinit, 0)

    def dscan(j, carry):
        w16 = wfull[pl.ds(j * L, L)]
        wg = lax.shift_right_logical(w16, 7)
        m = jnp.logical_and(w16 >= 0, (wg & (NW - 1)) == wid)
        idx = lax.shift_right_logical(wg, 5) * 128 + (w16 & 127)
        idx = jnp.where(m, idx, 0)
        plsc.store_scatter(dvm, [idx], j * L + iota, mask=m)
        return carry

    lax.fori_loop(0, NUM_GRIDS // L, dscan, 0)

    rows16 = [iota + 16 * b for b in range(C // L)]

    def fire(k, buf, sem):
        col = (k * NW + wid) * 128

        @pl.when(col < NUM_POINTS)
        def _():
            for g in range(C // 8):
                pltpu.async_copy(
                    featT_hbm.at[pl.ds(8 * g, 8), pl.ds(col, 128)],
                    buf.at[pl.ds(8 * g, 8)], sem)

    def drain(k, buf, sem):
        @pl.when((k * NW + wid) * 128 < NUM_POINTS)
        def _():
            pltpu.make_async_copy(featT_hbm.at[pl.ds(0, C), pl.ds(0, 128)],
                                  buf, sem).wait()

    def process(w, buf):
        def jv_body(jv, wcnt):
            dvec = dvm[pl.ds(w * 128 + jv * L, L)]
            mask = dvec >= 0

            def cond(state):
                m, _ = state
                return jnp.any(m)

            def body(state):
                m, cnt = state
                lsp = plsc.all_reduce_ffs(m)
                pidx = w * 128 + jv * L + lsp
                v = plsc.load_gather(dvm, [pidx])[0]
                pcol = jv * L + lsp
                for b in range(C // L):
                    vals = plsc.load_gather(buf, [rows16[b], pcol])
                    plsc.store_scatter(
                        slots, [cnt + iota * 0, 16 * b + iota], vals)
                pltpu.async_copy(slots.at[pl.ds(cnt, 1)],
                                 g_hbm.at[pl.ds(v, 1)], gsem)
                return jnp.logical_and(m, iota != lsp), cnt + 1

            m2, wcnt = lax.while_loop(cond, body, (mask, wcnt))
            return wcnt

        wcnt = lax.fori_loop(0, 128 // L, jv_body, 0)

        def gdrain(i, carry):
            pltpu.make_async_copy(
                winner_hbm.at[pl.ds(0, C)], wfull.at[pl.ds(0, C)],
                gsem).wait()
            return carry

        lax.fori_loop(0, wcnt, gdrain, 0)

    fire(0, bufa, sema)

    def wloop(j, carry):
        fire(2 * j + 1, bufb, semb)
        drain(2 * j, bufa, sema)
        process(2 * j, bufa)
        fire(2 * j + 2, bufa, sema)
        drain(2 * j + 1, bufb, semb)
        process(2 * j + 1, bufb)
        return carry

    lax.fori_loop(0, (NWIN - 1) // 2, wloop, 0)
    drain(NWIN - 1, bufa, sema)
    process(NWIN - 1, bufa)


def _k3_body(g_ref, w_ref, o_ref):
    g = g_ref[...]
    w = w_ref[0]
    gt = jnp.transpose(g, (1, 0))
    o_ref[...] = jnp.where(w >= 0, gt, jnp.float32(0.0))


_NBLK = 16
_BV = NUM_GRIDS // _NBLK

_k3_transpose = pl.pallas_call(
    _k3_body,
    grid=(_NBLK,),
    in_specs=[
        pl.BlockSpec((_BV, C), lambda i: (i, 0)),
        pl.BlockSpec((1, 1, _BV), lambda i: (i, 0, 0)),
    ],
    out_specs=pl.BlockSpec((C, _BV), lambda i: (0, i)),
    out_shape=jax.ShapeDtypeStruct((C, NUM_GRIDS), jnp.float32),
)


@jax.jit
def kernel(feat, bev_feat, lidar_coor_1d):
    del bev_feat
    coor = lidar_coor_1d.astype(jnp.int32)
    pad = jnp.full((NPAD - NUM_POINTS,), NUM_GRIDS, jnp.int32)
    coor = jnp.concatenate([coor, pad])
    winners = _k1_scatter_max(coor)
    winner = _k2a_merge(winners)
    g = _k2b_stream_select(winner, jnp.transpose(feat))
    out = _k3_transpose(g, winner.reshape(_NBLK, 1, _BV))
    return out.reshape(1, C, GY, GX)


# speedup vs baseline: 3.5497x; 1.0858x over previous
import functools

import jax
import jax.numpy as jnp
from jax import lax
from jax.experimental import pallas as pl
from jax.experimental.pallas import tpu as pltpu
from jax.experimental.pallas import tpu_sc as plsc

C = 80
GY, GX = 128, 128
NUM_POINTS = 249216
NUM_GRIDS = GY * GX

NC, NS, L = 2, 16, 16
NW = NC * NS
UNROLL = 4
CHUNK = 7808
NPAD = NW * CHUNK
NVREG = CHUNK // L
PRIV = 16512
VPT = NUM_GRIDS // NW

_mesh = plsc.VectorSubcoreMesh(
    core_axis_name="c", subcore_axis_name="s", num_cores=NC, num_subcores=NS
)


def _wid():
    return lax.axis_index("s") * NC + lax.axis_index("c")


@functools.partial(
    pl.kernel,
    out_type=jax.ShapeDtypeStruct((NW * NUM_GRIDS,), jnp.int32),
    mesh=_mesh,
    scratch_types=[
        pltpu.VMEM((CHUNK,), jnp.int32),
        pltpu.VMEM((PRIV,), jnp.int32),
        pltpu.VMEM((UNROLL * L,), jnp.int32),
    ],
    compiler_params=pltpu.CompilerParams(needs_layout_passes=False),
)
def _k1_scatter_max(coor_hbm, win_hbm, cvm, priv, sh):
    wid = _wid()
    base = wid * CHUNK
    pltpu.sync_copy(coor_hbm.at[pl.ds(base, CHUNK)], cvm)

    iota = lax.iota(jnp.int32, L)
    neg1 = jnp.full((L,), -1, jnp.int32)

    def init_body(i, carry):
        for u in range(8):
            priv[pl.ds((i * 8 + u) * L, L)] = neg1
        return carry

    lax.fori_loop(0, PRIV // (8 * L), init_body, 0)

    shm1 = jnp.maximum(iota - 1, 0)
    mge1 = iota >= 1
    is_last_lane = iota == L - 1

    def body(k, carry):
        for u in range(UNROLL):
            kk = k * UNROLL + u
            c16 = cvm[pl.ds(kk * L, L)]
            gidx = base + kk * L + iota
            key = c16 * L + iota
            ks, vs = plsc.sort_key_val(key, gidx)
            cs = lax.shift_right_logical(ks, 4)
            plsc.store_scatter(sh, [shm1 + u * L], cs, mask=mge1)
            nxt = sh[pl.ds(u * L, L)]
            last = jnp.logical_or(cs != nxt, is_last_lane)
            plsc.store_scatter(priv, [cs], vs, mask=last)
        return carry

    lax.fori_loop(0, NVREG // UNROLL, body, 0)
    pltpu.sync_copy(priv.at[pl.ds(0, NUM_GRIDS)],
                    win_hbm.at[pl.ds(wid * NUM_GRIDS, NUM_GRIDS)])


@functools.partial(
    pl.kernel,
    out_type=(
        jax.ShapeDtypeStruct((NUM_GRIDS, C), jnp.float32),
        jax.ShapeDtypeStruct((NUM_GRIDS,), jnp.int32),
    ),
    mesh=_mesh,
    scratch_types=[
        pltpu.VMEM((NW * VPT,), jnp.int32),
        pltpu.VMEM((VPT,), jnp.int32),
        pltpu.VMEM((VPT, C), jnp.float32),
        pltpu.SemaphoreType.DMA,
        pltpu.SemaphoreType.DMA,
    ],
    compiler_params=pltpu.CompilerParams(needs_layout_passes=False),
)
def _k2_merge_gather(win_hbm, feat_hbm, g_hbm, winner_hbm, slab, wvm,
                     rows, sem, gsem):
    wid = _wid()
    vbase = wid * VPT

    for t in range(NW):
        pltpu.async_copy(win_hbm.at[pl.ds(t * NUM_GRIDS + vbase, VPT)],
                         slab.at[pl.ds(t * VPT, VPT)], sem)
    pltpu.make_async_copy(win_hbm.at[pl.ds(0, NW * VPT)], slab, sem).wait()

    def merge_body(j, carry):
        m = slab[pl.ds(j * L, L)]
        for t in range(1, NW):
            m = jnp.maximum(m, slab[pl.ds(t * VPT + j * L, L)])
        wvm[pl.ds(j * L, L)] = m
        return carry

    lax.fori_loop(0, VPT // L, merge_body, 0)

    def gather_body(j, carry):
        iv = jnp.maximum(wvm[pl.ds(j * L, L)], 0)
        for q in range(L):
            pltpu.async_copy(feat_hbm.at[pl.ds(iv[q], 1)],
                             rows.at[pl.ds(j * L + q, 1)], gsem)
        return carry

    lax.fori_loop(0, VPT // L, gather_body, 0)
    pltpu.make_async_copy(feat_hbm.at[pl.ds(0, VPT)], rows, gsem).wait()

    pltpu.sync_copy(rows, g_hbm.at[pl.ds(vbase, VPT)])
    pltpu.sync_copy(wvm, winner_hbm.at[pl.ds(vbase, VPT)])


def _k3_body(g_ref, w_ref, o_ref):
    g = g_ref[...]
    w = w_ref[0]
    gt = jnp.transpose(g, (1, 0))
    o_ref[...] = jnp.where(w >= 0, gt, jnp.float32(0.0))


_NBLK = 16
_BV = NUM_GRIDS // _NBLK

_k3_transpose = pl.pallas_call(
    _k3_body,
    grid=(_NBLK,),
    in_specs=[
        pl.BlockSpec((_BV, C), lambda i: (i, 0)),
        pl.BlockSpec((1, 1, _BV), lambda i: (i, 0, 0)),
    ],
    out_specs=pl.BlockSpec((C, _BV), lambda i: (0, i)),
    out_shape=jax.ShapeDtypeStruct((C, NUM_GRIDS), jnp.float32),
)


@jax.jit
def kernel(feat, bev_feat, lidar_coor_1d):
    del bev_feat
    coor = lidar_coor_1d.astype(jnp.int32)
    pad = jnp.full((NPAD - NUM_POINTS,), NUM_GRIDS, jnp.int32)
    coor = jnp.concatenate([coor, pad])
    winners = _k1_scatter_max(coor)
    g, winner = _k2_merge_gather(winners, feat)
    out = _k3_transpose(g, winner.reshape(_NBLK, 1, _BV))
    return out.reshape(1, C, GY, GX)
